# Initial kernel scaffold; baseline (speedup 1.0000x reference)
import jax
import jax.numpy as jnp
from jax.experimental import pallas as pl


def kernel(x, edge_index, W1, b1, W2, b2):
    raise NotImplementedError("write your pallas kernel here")


# speedup vs baseline: 30.8208x; 30.8208x over previous
import functools

import jax
import jax.numpy as jnp
from jax import lax
from jax.experimental import pallas as pl
from jax.experimental.pallas import tpu as pltpu
from jax.experimental.pallas import tpu_sc as plsc

NC = 2
NS = 16
L = 128
NT = NC * NS

_mesh = functools.partial(
    plsc.VectorSubcoreMesh,
    core_axis_name="c",
    subcore_axis_name="s",
    num_cores=NC,
    num_subcores=NS,
)


def _make_deg_kernel(n_pad, nchunk, stripe, width):
  @functools.partial(
      pl.kernel,
      mesh=_mesh(),
      out_type=jax.ShapeDtypeStruct((NC, n_pad, width), jnp.float32),
      scratch_types=[
          pltpu.VMEM((nchunk, L), jnp.int32),
          pltpu.VMEM((L, width), jnp.float32),
          pltpu.VMEM((stripe, width), jnp.float32),
          pltpu.VMEM_SHARED((n_pad, width), jnp.float32),
          pltpu.SemaphoreType.DMA,
      ],
      compiler_params=pltpu.CompilerParams(use_tc_tiling_on_sc=False),
  )
  def deg_kernel(cidx_hbm, ones_hbm, zeros_hbm, out_hbm,
                 cidx_v, ones_v, zer_v, acc, sem):
    cid = lax.axis_index("c")
    sid = lax.axis_index("s")
    wid = sid * NC + cid
    pltpu.sync_copy(cidx_hbm.at[wid], cidx_v)
    pltpu.sync_copy(ones_hbm, ones_v)
    pltpu.sync_copy(zeros_hbm, zer_v)
    pltpu.sync_copy(zer_v, acc.at[pl.ds(sid * stripe, stripe)])
    plsc.subcore_barrier()

    def body(j, carry):
      pltpu.sync_copy(ones_v, acc.at[cidx_v.at[j]], add=True)
      return carry

    lax.fori_loop(0, nchunk, body, 0)
    plsc.subcore_barrier()
    pltpu.sync_copy(acc.at[pl.ds(sid * stripe, stripe)],
                    out_hbm.at[cid, pl.ds(sid * stripe, stripe)])

  return deg_kernel


def _make_agg_kernel(n_pad, nchunk, stripe, width):
  @functools.partial(
      pl.kernel,
      mesh=_mesh(),
      out_type=jax.ShapeDtypeStruct((NC, n_pad, width), jnp.float32),
      scratch_types=[
          pltpu.VMEM((nchunk, L), jnp.int32),
          pltpu.VMEM((nchunk, L), jnp.int32),
          pltpu.VMEM((L, width), jnp.float32),
          pltpu.VMEM((stripe, width), jnp.float32),
          pltpu.VMEM_SHARED((n_pad, width), jnp.float32),
          pltpu.SemaphoreType.DMA,
      ],
      compiler_params=pltpu.CompilerParams(use_tc_tiling_on_sc=False),
  )
  def agg_kernel(table_hbm, ridx_hbm, cidx_hbm, zeros_hbm, out_hbm,
                 ridx_v, cidx_v, buf, zer_v, acc, sem):
    cid = lax.axis_index("c")
    sid = lax.axis_index("s")
    wid = sid * NC + cid
    pltpu.sync_copy(ridx_hbm.at[wid], ridx_v)
    pltpu.sync_copy(cidx_hbm.at[wid], cidx_v)
    pltpu.sync_copy(zeros_hbm, zer_v)
    pltpu.sync_copy(zer_v, acc.at[pl.ds(sid * stripe, stripe)])
    plsc.subcore_barrier()

    def body(j, carry):
      pltpu.async_copy(table_hbm.at[ridx_v.at[j]], buf, sem).wait()
      pltpu.sync_copy(buf, acc.at[cidx_v.at[j]], add=True)
      return carry

    lax.fori_loop(0, nchunk, body, 0)
    plsc.subcore_barrier()
    pltpu.sync_copy(acc.at[pl.ds(sid * stripe, stripe)],
                    out_hbm.at[cid, pl.ds(sid * stripe, stripe)])

  return agg_kernel


def _tc_a_body(deg_ref, x_ref, w1_ref, hs_ref, dis_ref):
  deg = deg_ref[0, :, :1] + deg_ref[1, :, :1] + 1.0
  dis = lax.rsqrt(deg)
  h = jnp.dot(x_ref[...], w1_ref[...], preferred_element_type=jnp.float32)
  hs_ref[...] = h * dis
  dis_ref[...] = dis


def _tc_b_body(p_ref, hs_ref, dis_ref, b1_ref, out_ref):
  dis = dis_ref[...]
  o1 = (p_ref[0] + p_ref[1] + hs_ref[...]) * dis + b1_ref[...]
  out_ref[...] = jnp.maximum(o1, 0.0) * dis


def _tc_c_body(p_ref, hs1_ref, dis_ref, w2_ref, b2_ref, out_ref):
  agg = (p_ref[0] + p_ref[1] + hs1_ref[...]) * dis_ref[...]
  o = jnp.dot(agg, w2_ref[...], preferred_element_type=jnp.float32)
  o = o + b2_ref[...]
  m = jnp.max(o, axis=1, keepdims=True)
  lse = jnp.log(jnp.sum(jnp.exp(o - m), axis=1, keepdims=True)) + m
  out_ref[...] = o - lse


def kernel(x, edge_index, W1, b1, W2, b2):
  n, d = x.shape
  hid = W1.shape[1]
  c = W2.shape[1]
  e = edge_index.shape[1]

  blk = 640
  n_pad = pl.cdiv(n + 1, blk) * blk
  stripe = n_pad // NS
  ept = pl.cdiv(e, NT)
  nchunk = pl.cdiv(ept, L)
  e_pad = NT * nchunk * L
  nblk = n_pad // blk

  row = jnp.pad(edge_index[0], (0, e_pad - e), constant_values=n)
  col = jnp.pad(edge_index[1], (0, e_pad - e), constant_values=n)
  ridx = row.reshape(NT, nchunk, L)
  cidx = col.reshape(NT, nchunk, L)
  x_p = jnp.pad(x, ((0, n_pad - n), (0, 0)))
  ones_l = jnp.ones((L, hid), jnp.float32)
  zeros_sh = jnp.zeros((stripe, hid), jnp.float32)

  deg_p = _make_deg_kernel(n_pad, nchunk, stripe, hid)(cidx, ones_l, zeros_sh)

  hs, dis = pl.pallas_call(
      _tc_a_body,
      grid=(nblk,),
      in_specs=[
          pl.BlockSpec((NC, blk, hid), lambda i: (0, i, 0)),
          pl.BlockSpec((blk, d), lambda i: (i, 0)),
          pl.BlockSpec((d, hid), lambda i: (0, 0)),
      ],
      out_specs=[
          pl.BlockSpec((blk, hid), lambda i: (i, 0)),
          pl.BlockSpec((blk, 1), lambda i: (i, 0)),
      ],
      out_shape=[
          jax.ShapeDtypeStruct((n_pad, hid), jnp.float32),
          jax.ShapeDtypeStruct((n_pad, 1), jnp.float32),
      ],
  )(deg_p, x_p, W1)

  agg = _make_agg_kernel(n_pad, nchunk, stripe, hid)
  p1 = agg(hs, ridx, cidx, zeros_sh)

  hs1 = pl.pallas_call(
      _tc_b_body,
      grid=(nblk,),
      in_specs=[
          pl.BlockSpec((NC, blk, hid), lambda i: (0, i, 0)),
          pl.BlockSpec((blk, hid), lambda i: (i, 0)),
          pl.BlockSpec((blk, 1), lambda i: (i, 0)),
          pl.BlockSpec((1, hid), lambda i: (0, 0)),
      ],
      out_specs=pl.BlockSpec((blk, hid), lambda i: (i, 0)),
      out_shape=jax.ShapeDtypeStruct((n_pad, hid), jnp.float32),
  )(p1, hs, dis, b1.reshape(1, hid))

  p2 = agg(hs1, ridx, cidx, zeros_sh)

  out = pl.pallas_call(
      _tc_c_body,
      grid=(nblk,),
      in_specs=[
          pl.BlockSpec((NC, blk, hid), lambda i: (0, i, 0)),
          pl.BlockSpec((blk, hid), lambda i: (i, 0)),
          pl.BlockSpec((blk, 1), lambda i: (i, 0)),
          pl.BlockSpec((hid, c), lambda i: (0, 0)),
          pl.BlockSpec((1, c), lambda i: (0, 0)),
      ],
      out_specs=pl.BlockSpec((blk, c), lambda i: (i, 0)),
      out_shape=jax.ShapeDtypeStruct((n_pad, c), jnp.float32),
  )(p2, hs1, dis, W2, b2.reshape(1, c))

  return out[:n]


# speedup vs baseline: 33.7435x; 1.0948x over previous
import functools

import jax
import jax.numpy as jnp
from jax import lax
from jax.experimental import pallas as pl
from jax.experimental.pallas import tpu as pltpu
from jax.experimental.pallas import tpu_sc as plsc

NC = 2
NS = 16
L = 128
NT = NC * NS

_mesh = functools.partial(
    plsc.VectorSubcoreMesh,
    core_axis_name="c",
    subcore_axis_name="s",
    num_cores=NC,
    num_subcores=NS,
)


def _make_deg_kernel(n_pad, nchunk, stripe, width):
  @functools.partial(
      pl.kernel,
      mesh=_mesh(),
      out_type=jax.ShapeDtypeStruct((NC, n_pad, width), jnp.float32),
      scratch_types=[
          pltpu.VMEM((nchunk, L), jnp.int32),
          pltpu.VMEM((L, width), jnp.float32),
          pltpu.VMEM((stripe, width), jnp.float32),
          pltpu.VMEM_SHARED((n_pad, width), jnp.float32),
          pltpu.SemaphoreType.DMA,
      ],
      compiler_params=pltpu.CompilerParams(use_tc_tiling_on_sc=False),
  )
  def deg_kernel(cidx_hbm, ones_hbm, zeros_hbm, out_hbm,
                 cidx_v, ones_v, zer_v, acc, sem):
    cid = lax.axis_index("c")
    sid = lax.axis_index("s")
    wid = sid * NC + cid
    pltpu.sync_copy(cidx_hbm.at[wid], cidx_v)
    pltpu.sync_copy(ones_hbm, ones_v)
    pltpu.sync_copy(zeros_hbm, zer_v)
    pltpu.sync_copy(zer_v, acc.at[pl.ds(sid * stripe, stripe)])
    plsc.subcore_barrier()

    def body(j, carry):
      pltpu.sync_copy(ones_v, acc.at[cidx_v.at[j]], add=True)
      return carry

    lax.fori_loop(0, nchunk, body, 0)
    plsc.subcore_barrier()
    pltpu.sync_copy(acc.at[pl.ds(sid * stripe, stripe)],
                    out_hbm.at[cid, pl.ds(sid * stripe, stripe)])

  return deg_kernel


def _make_agg_kernel(n_pad, nchunk, stripe, width):
  @functools.partial(
      pl.kernel,
      mesh=_mesh(),
      out_type=jax.ShapeDtypeStruct((NC, n_pad, width), jnp.float32),
      scratch_types=[
          pltpu.VMEM((nchunk, L), jnp.int32),
          pltpu.VMEM((nchunk, L), jnp.int32),
          pltpu.VMEM((L, width), jnp.float32),
          pltpu.VMEM((L, width), jnp.float32),
          pltpu.VMEM((stripe, width), jnp.float32),
          pltpu.VMEM_SHARED((n_pad, width), jnp.float32),
          pltpu.SemaphoreType.DMA,
          pltpu.SemaphoreType.DMA,
      ],
      compiler_params=pltpu.CompilerParams(use_tc_tiling_on_sc=False),
  )
  def agg_kernel(table_hbm, ridx_hbm, cidx_hbm, zeros_hbm, out_hbm,
                 ridx_v, cidx_v, buf0, buf1, zer_v, acc, sem0, sem1):
    cid = lax.axis_index("c")
    sid = lax.axis_index("s")
    wid = sid * NC + cid
    pltpu.sync_copy(ridx_hbm.at[wid], ridx_v)
    pltpu.sync_copy(cidx_hbm.at[wid], cidx_v)
    pltpu.sync_copy(zeros_hbm, zer_v)
    pltpu.sync_copy(zer_v, acc.at[pl.ds(sid * stripe, stripe)])
    plsc.subcore_barrier()

    pltpu.async_copy(table_hbm.at[ridx_v.at[0]], buf0, sem0)

    def body(k, carry):
      j0 = 2 * k
      j1 = j0 + 1
      pltpu.make_async_copy(table_hbm.at[pl.ds(0, L)], buf0, sem0).wait()
      pltpu.async_copy(table_hbm.at[ridx_v.at[j1]], buf1, sem1)
      pltpu.sync_copy(buf0, acc.at[cidx_v.at[j0]], add=True)
      jn = jnp.minimum(j0 + 2, nchunk - 2)
      pltpu.async_copy(table_hbm.at[ridx_v.at[jn]], buf0, sem0)
      pltpu.make_async_copy(table_hbm.at[pl.ds(0, L)], buf1, sem1).wait()
      pltpu.sync_copy(buf1, acc.at[cidx_v.at[j1]], add=True)
      return carry

    lax.fori_loop(0, nchunk // 2, body, 0)
    pltpu.make_async_copy(table_hbm.at[pl.ds(0, L)], buf0, sem0).wait()
    plsc.subcore_barrier()
    pltpu.sync_copy(acc.at[pl.ds(sid * stripe, stripe)],
                    out_hbm.at[cid, pl.ds(sid * stripe, stripe)])

  return agg_kernel


def _tc_a_body(deg_ref, x_ref, w1_ref, hs_ref, dis_ref):
  deg = deg_ref[0, :, :1] + deg_ref[1, :, :1] + 1.0
  dis = lax.rsqrt(deg)
  h = jnp.dot(x_ref[...], w1_ref[...], preferred_element_type=jnp.float32)
  hs_ref[...] = h * dis
  dis_ref[...] = dis


def _tc_b_body(p_ref, hs_ref, dis_ref, b1_ref, out_ref):
  dis = dis_ref[...]
  o1 = (p_ref[0] + p_ref[1] + hs_ref[...]) * dis + b1_ref[...]
  out_ref[...] = jnp.maximum(o1, 0.0) * dis


def _tc_c_body(p_ref, hs1_ref, dis_ref, w2_ref, b2_ref, out_ref):
  agg = (p_ref[0] + p_ref[1] + hs1_ref[...]) * dis_ref[...]
  o = jnp.dot(agg, w2_ref[...], preferred_element_type=jnp.float32)
  o = o + b2_ref[...]
  m = jnp.max(o, axis=1, keepdims=True)
  lse = jnp.log(jnp.sum(jnp.exp(o - m), axis=1, keepdims=True)) + m
  out_ref[...] = o - lse


def kernel(x, edge_index, W1, b1, W2, b2):
  n, d = x.shape
  hid = W1.shape[1]
  c = W2.shape[1]
  e = edge_index.shape[1]

  blk = 640
  n_pad = pl.cdiv(n + 1, blk) * blk
  stripe = n_pad // NS
  ept = pl.cdiv(e, NT)
  nchunk = pl.cdiv(ept, L)
  nchunk += nchunk % 2
  e_pad = NT * nchunk * L
  nblk = n_pad // blk

  row = jnp.pad(edge_index[0], (0, e_pad - e), constant_values=n)
  col = jnp.pad(edge_index[1], (0, e_pad - e), constant_values=n)
  ridx = row.reshape(NT, nchunk, L)
  cidx = col.reshape(NT, nchunk, L)
  x_p = jnp.pad(x, ((0, n_pad - n), (0, 0)))
  ones_l = jnp.ones((L, hid), jnp.float32)
  zeros_sh = jnp.zeros((stripe, hid), jnp.float32)

  deg_p = _make_deg_kernel(n_pad, nchunk, stripe, hid)(cidx, ones_l, zeros_sh)

  hs, dis = pl.pallas_call(
      _tc_a_body,
      grid=(nblk,),
      in_specs=[
          pl.BlockSpec((NC, blk, hid), lambda i: (0, i, 0)),
          pl.BlockSpec((blk, d), lambda i: (i, 0)),
          pl.BlockSpec((d, hid), lambda i: (0, 0)),
      ],
      out_specs=[
          pl.BlockSpec((blk, hid), lambda i: (i, 0)),
          pl.BlockSpec((blk, 1), lambda i: (i, 0)),
      ],
      out_shape=[
          jax.ShapeDtypeStruct((n_pad, hid), jnp.float32),
          jax.ShapeDtypeStruct((n_pad, 1), jnp.float32),
      ],
  )(deg_p, x_p, W1)

  agg = _make_agg_kernel(n_pad, nchunk, stripe, hid)
  p1 = agg(hs, ridx, cidx, zeros_sh)

  hs1 = pl.pallas_call(
      _tc_b_body,
      grid=(nblk,),
      in_specs=[
          pl.BlockSpec((NC, blk, hid), lambda i: (0, i, 0)),
          pl.BlockSpec((blk, hid), lambda i: (i, 0)),
          pl.BlockSpec((blk, 1), lambda i: (i, 0)),
          pl.BlockSpec((1, hid), lambda i: (0, 0)),
      ],
      out_specs=pl.BlockSpec((blk, hid), lambda i: (i, 0)),
      out_shape=jax.ShapeDtypeStruct((n_pad, hid), jnp.float32),
  )(p1, hs, dis, b1.reshape(1, hid))

  p2 = agg(hs1, ridx, cidx, zeros_sh)

  out = pl.pallas_call(
      _tc_c_body,
      grid=(nblk,),
      in_specs=[
          pl.BlockSpec((NC, blk, hid), lambda i: (0, i, 0)),
          pl.BlockSpec((blk, hid), lambda i: (i, 0)),
          pl.BlockSpec((blk, 1), lambda i: (i, 0)),
          pl.BlockSpec((hid, c), lambda i: (0, 0)),
          pl.BlockSpec((1, c), lambda i: (0, 0)),
      ],
      out_specs=pl.BlockSpec((blk, c), lambda i: (i, 0)),
      out_shape=jax.ShapeDtypeStruct((n_pad, c), jnp.float32),
  )(p2, hs1, dis, W2, b2.reshape(1, c))

  return out[:n]


# speedup vs baseline: 33.7481x; 1.0001x over previous
import functools

import jax
import jax.numpy as jnp
from jax import lax
from jax.experimental import pallas as pl
from jax.experimental.pallas import tpu as pltpu
from jax.experimental.pallas import tpu_sc as plsc

NC = 2
NS = 16
L = 512
NT = NC * NS

_mesh = functools.partial(
    plsc.VectorSubcoreMesh,
    core_axis_name="c",
    subcore_axis_name="s",
    num_cores=NC,
    num_subcores=NS,
)


def _make_deg_kernel(n_pad, nchunk, stripe, width):
  @functools.partial(
      pl.kernel,
      mesh=_mesh(),
      out_type=jax.ShapeDtypeStruct((NC, n_pad, width), jnp.float32),
      scratch_types=[
          pltpu.VMEM((nchunk, L), jnp.int32),
          pltpu.VMEM((L, width), jnp.float32),
          pltpu.VMEM((stripe, width), jnp.float32),
          pltpu.VMEM_SHARED((n_pad, width), jnp.float32),
          pltpu.SemaphoreType.DMA,
      ],
      compiler_params=pltpu.CompilerParams(use_tc_tiling_on_sc=False),
  )
  def deg_kernel(cidx_hbm, ones_hbm, zeros_hbm, out_hbm,
                 cidx_v, ones_v, zer_v, acc, sem):
    cid = lax.axis_index("c")
    sid = lax.axis_index("s")
    wid = sid * NC + cid
    pltpu.sync_copy(cidx_hbm.at[wid], cidx_v)
    pltpu.sync_copy(ones_hbm, ones_v)
    pltpu.sync_copy(zeros_hbm, zer_v)
    pltpu.sync_copy(zer_v, acc.at[pl.ds(sid * stripe, stripe)])
    plsc.subcore_barrier()

    def body(j, carry):
      pltpu.sync_copy(ones_v, acc.at[cidx_v.at[j]], add=True)
      return carry

    lax.fori_loop(0, nchunk, body, 0)
    plsc.subcore_barrier()
    pltpu.sync_copy(acc.at[pl.ds(sid * stripe, stripe)],
                    out_hbm.at[cid, pl.ds(sid * stripe, stripe)])

  return deg_kernel


def _make_agg_kernel(n_pad, nchunk, stripe, width):
  @functools.partial(
      pl.kernel,
      mesh=_mesh(),
      out_type=jax.ShapeDtypeStruct((NC, n_pad, width), jnp.float32),
      scratch_types=[
          pltpu.VMEM((nchunk, L), jnp.int32),
          pltpu.VMEM((nchunk, L), jnp.int32),
          pltpu.VMEM((L, width), jnp.float32),
          pltpu.VMEM((L, width), jnp.float32),
          pltpu.VMEM((stripe, width), jnp.float32),
          pltpu.VMEM_SHARED((n_pad, width), jnp.float32),
          pltpu.SemaphoreType.DMA,
          pltpu.SemaphoreType.DMA,
      ],
      compiler_params=pltpu.CompilerParams(use_tc_tiling_on_sc=False),
  )
  def agg_kernel(table_hbm, ridx_hbm, cidx_hbm, zeros_hbm, out_hbm,
                 ridx_v, cidx_v, buf0, buf1, zer_v, acc, sem0, sem1):
    cid = lax.axis_index("c")
    sid = lax.axis_index("s")
    wid = sid * NC + cid
    pltpu.sync_copy(ridx_hbm.at[wid], ridx_v)
    pltpu.sync_copy(cidx_hbm.at[wid], cidx_v)
    pltpu.sync_copy(zeros_hbm, zer_v)
    pltpu.sync_copy(zer_v, acc.at[pl.ds(sid * stripe, stripe)])
    plsc.subcore_barrier()

    pltpu.async_copy(table_hbm.at[ridx_v.at[0]], buf0, sem0)

    def body(k, carry):
      j0 = 2 * k
      j1 = j0 + 1
      pltpu.make_async_copy(table_hbm.at[pl.ds(0, L)], buf0, sem0).wait()
      pltpu.async_copy(table_hbm.at[ridx_v.at[j1]], buf1, sem1)
      pltpu.sync_copy(buf0, acc.at[cidx_v.at[j0]], add=True)
      jn = jnp.minimum(j0 + 2, nchunk - 2)
      pltpu.async_copy(table_hbm.at[ridx_v.at[jn]], buf0, sem0)
      pltpu.make_async_copy(table_hbm.at[pl.ds(0, L)], buf1, sem1).wait()
      pltpu.sync_copy(buf1, acc.at[cidx_v.at[j1]], add=True)
      return carry

    lax.fori_loop(0, nchunk // 2, body, 0)
    pltpu.make_async_copy(table_hbm.at[pl.ds(0, L)], buf0, sem0).wait()
    plsc.subcore_barrier()
    pltpu.sync_copy(acc.at[pl.ds(sid * stripe, stripe)],
                    out_hbm.at[cid, pl.ds(sid * stripe, stripe)])

  return agg_kernel


def _tc_a_body(deg_ref, x_ref, w1_ref, hs_ref, dis_ref):
  deg = deg_ref[0, :, :1] + deg_ref[1, :, :1] + 1.0
  dis = lax.rsqrt(deg)
  h = jnp.dot(x_ref[...], w1_ref[...], preferred_element_type=jnp.float32)
  hs_ref[...] = h * dis
  dis_ref[...] = dis


def _tc_b_body(p_ref, hs_ref, dis_ref, b1_ref, out_ref):
  dis = dis_ref[...]
  o1 = (p_ref[0] + p_ref[1] + hs_ref[...]) * dis + b1_ref[...]
  out_ref[...] = jnp.maximum(o1, 0.0) * dis


def _tc_c_body(p_ref, hs1_ref, dis_ref, w2_ref, b2_ref, out_ref):
  agg = (p_ref[0] + p_ref[1] + hs1_ref[...]) * dis_ref[...]
  o = jnp.dot(agg, w2_ref[...], preferred_element_type=jnp.float32)
  o = o + b2_ref[...]
  m = jnp.max(o, axis=1, keepdims=True)
  lse = jnp.log(jnp.sum(jnp.exp(o - m), axis=1, keepdims=True)) + m
  out_ref[...] = o - lse


def kernel(x, edge_index, W1, b1, W2, b2):
  n, d = x.shape
  hid = W1.shape[1]
  c = W2.shape[1]
  e = edge_index.shape[1]

  blk = 640
  n_pad = pl.cdiv(n + 1, blk) * blk
  stripe = n_pad // NS
  ept = pl.cdiv(e, NT)
  nchunk = pl.cdiv(ept, L)
  nchunk += nchunk % 2
  e_pad = NT * nchunk * L
  nblk = n_pad // blk

  row = jnp.pad(edge_index[0], (0, e_pad - e), constant_values=n)
  col = jnp.pad(edge_index[1], (0, e_pad - e), constant_values=n)
  ridx = row.reshape(NT, nchunk, L)
  cidx = col.reshape(NT, nchunk, L)
  x_p = jnp.pad(x, ((0, n_pad - n), (0, 0)))
  ones_l = jnp.ones((L, 8), jnp.float32)
  zeros_s8 = jnp.zeros((stripe, 8), jnp.float32)
  zeros_sh = jnp.zeros((stripe, hid), jnp.float32)

  deg_p = _make_deg_kernel(n_pad, nchunk, stripe, 8)(cidx, ones_l, zeros_s8)

  hs, dis = pl.pallas_call(
      _tc_a_body,
      grid=(nblk,),
      in_specs=[
          pl.BlockSpec((NC, blk, 8), lambda i: (0, i, 0)),
          pl.BlockSpec((blk, d), lambda i: (i, 0)),
          pl.BlockSpec((d, hid), lambda i: (0, 0)),
      ],
      out_specs=[
          pl.BlockSpec((blk, hid), lambda i: (i, 0)),
          pl.BlockSpec((blk, 1), lambda i: (i, 0)),
      ],
      out_shape=[
          jax.ShapeDtypeStruct((n_pad, hid), jnp.float32),
          jax.ShapeDtypeStruct((n_pad, 1), jnp.float32),
      ],
  )(deg_p, x_p, W1)

  agg = _make_agg_kernel(n_pad, nchunk, stripe, hid)
  p1 = agg(hs, ridx, cidx, zeros_sh)

  hs1 = pl.pallas_call(
      _tc_b_body,
      grid=(nblk,),
      in_specs=[
          pl.BlockSpec((NC, blk, hid), lambda i: (0, i, 0)),
          pl.BlockSpec((blk, hid), lambda i: (i, 0)),
          pl.BlockSpec((blk, 1), lambda i: (i, 0)),
          pl.BlockSpec((1, hid), lambda i: (0, 0)),
      ],
      out_specs=pl.BlockSpec((blk, hid), lambda i: (i, 0)),
      out_shape=jax.ShapeDtypeStruct((n_pad, hid), jnp.float32),
  )(p1, hs, dis, b1.reshape(1, hid))

  p2 = agg(hs1, ridx, cidx, zeros_sh)

  out = pl.pallas_call(
      _tc_c_body,
      grid=(nblk,),
      in_specs=[
          pl.BlockSpec((NC, blk, hid), lambda i: (0, i, 0)),
          pl.BlockSpec((blk, hid), lambda i: (i, 0)),
          pl.BlockSpec((blk, 1), lambda i: (i, 0)),
          pl.BlockSpec((hid, c), lambda i: (0, 0)),
          pl.BlockSpec((1, c), lambda i: (0, 0)),
      ],
      out_specs=pl.BlockSpec((blk, c), lambda i: (i, 0)),
      out_shape=jax.ShapeDtypeStruct((n_pad, c), jnp.float32),
  )(p2, hs1, dis, W2, b2.reshape(1, c))

  return out[:n]


# speedup vs baseline: 52.6240x; 1.5593x over previous
import functools

import jax
import jax.numpy as jnp
from jax import lax
from jax.experimental import pallas as pl
from jax.experimental.pallas import tpu as pltpu
from jax.experimental.pallas import tpu_sc as plsc

NC = 2
NS = 16
L = 512
NT = NC * NS

_mesh = functools.partial(
    plsc.VectorSubcoreMesh,
    core_axis_name="c",
    subcore_axis_name="s",
    num_cores=NC,
    num_subcores=NS,
)
_sc_params = pltpu.CompilerParams(use_tc_tiling_on_sc=False)


def _agg_loop(tbl, ridx_v, cidx_v, acc, buf0, buf1, sem0, sem1, nchunk):
  pltpu.async_copy(tbl.at[ridx_v.at[0]], buf0, sem0)

  def body(k, carry):
    j0 = 2 * k
    j1 = j0 + 1
    pltpu.make_async_copy(tbl.at[pl.ds(0, L)], buf0, sem0).wait()
    pltpu.async_copy(tbl.at[ridx_v.at[j1]], buf1, sem1)
    pltpu.sync_copy(buf0, acc.at[cidx_v.at[j0]], add=True)
    jn = jnp.minimum(j0 + 2, nchunk - 2)
    pltpu.async_copy(tbl.at[ridx_v.at[jn]], buf0, sem0)
    pltpu.make_async_copy(tbl.at[pl.ds(0, L)], buf1, sem1).wait()
    pltpu.sync_copy(buf1, acc.at[cidx_v.at[j1]], add=True)
    return carry

  lax.fori_loop(0, nchunk // 2, body, 0)
  pltpu.make_async_copy(tbl.at[pl.ds(0, L)], buf0, sem0).wait()


def _make_deg_kernel(n_pad, nchunk, stripe, width):
  @functools.partial(
      pl.kernel,
      mesh=_mesh(),
      out_type=jax.ShapeDtypeStruct((NC, n_pad, width), jnp.float32),
      scratch_types=[
          pltpu.VMEM((nchunk, L), jnp.int32),
          pltpu.VMEM((L, width), jnp.float32),
          pltpu.VMEM((stripe, width), jnp.float32),
          pltpu.VMEM_SHARED((n_pad, width), jnp.float32),
          pltpu.SemaphoreType.DMA,
      ],
      compiler_params=_sc_params,
  )
  def deg_kernel(cidx_hbm, ones_hbm, zeros_hbm, out_hbm,
                 cidx_v, ones_v, zer_v, acc, sem):
    cid = lax.axis_index("c")
    sid = lax.axis_index("s")
    wid = sid * NC + cid
    pltpu.sync_copy(cidx_hbm.at[wid], cidx_v)
    pltpu.sync_copy(ones_hbm, ones_v)
    pltpu.sync_copy(zeros_hbm, zer_v)
    pltpu.sync_copy(zer_v, acc.at[pl.ds(sid * stripe, stripe)])
    plsc.subcore_barrier()

    def body(j, carry):
      pltpu.sync_copy(ones_v, acc.at[cidx_v.at[j]], add=True)
      return carry

    lax.fori_loop(0, nchunk, body, 0)
    plsc.subcore_barrier()
    pltpu.sync_copy(acc.at[pl.ds(sid * stripe, stripe)],
                    out_hbm.at[cid, pl.ds(sid * stripe, stripe)])

  return deg_kernel


def _make_agg1_kernel(n_pad, nchunk, stripe, width):
  @functools.partial(
      pl.kernel,
      mesh=_mesh(),
      out_type=jax.ShapeDtypeStruct((NC, n_pad, width), jnp.float32),
      scratch_types=[
          pltpu.VMEM((nchunk, L), jnp.int32),
          pltpu.VMEM((nchunk, L), jnp.int32),
          pltpu.VMEM((L, width), jnp.float32),
          pltpu.VMEM((L, width), jnp.float32),
          pltpu.VMEM((stripe, width), jnp.float32),
          pltpu.VMEM_SHARED((n_pad, width), jnp.float32),
          pltpu.VMEM_SHARED((n_pad, width), jnp.float32),
          pltpu.SemaphoreType.DMA,
          pltpu.SemaphoreType.DMA,
      ],
      compiler_params=_sc_params,
  )
  def agg1_kernel(table_hbm, ridx_hbm, cidx_hbm, zeros_hbm, out_hbm,
                  ridx_v, cidx_v, buf0, buf1, zer_v, tbl, acc, sem0, sem1):
    cid = lax.axis_index("c")
    sid = lax.axis_index("s")
    wid = sid * NC + cid
    base = sid * stripe
    pltpu.sync_copy(ridx_hbm.at[wid], ridx_v)
    pltpu.sync_copy(cidx_hbm.at[wid], cidx_v)
    pltpu.sync_copy(zeros_hbm, zer_v)
    pltpu.sync_copy(zer_v, acc.at[pl.ds(base, stripe)])
    pltpu.sync_copy(table_hbm.at[pl.ds(base, stripe)],
                    tbl.at[pl.ds(base, stripe)])
    plsc.subcore_barrier()
    _agg_loop(tbl, ridx_v, cidx_v, acc, buf0, buf1, sem0, sem1, nchunk)
    plsc.subcore_barrier()
    pltpu.sync_copy(acc.at[pl.ds(base, stripe)],
                    out_hbm.at[cid, pl.ds(base, stripe)])

  return agg1_kernel


def _make_agg2_kernel(n_pad, nchunk, stripe, width):
  @functools.partial(
      pl.kernel,
      mesh=_mesh(),
      out_type=(
          jax.ShapeDtypeStruct((NC, n_pad, width), jnp.float32),
          jax.ShapeDtypeStruct((n_pad, width), jnp.float32),
      ),
      scratch_types=[
          pltpu.VMEM((nchunk, L), jnp.int32),
          pltpu.VMEM((nchunk, L), jnp.int32),
          pltpu.VMEM((L, width), jnp.float32),
          pltpu.VMEM((L, width), jnp.float32),
          pltpu.VMEM((stripe, width), jnp.float32),
          pltpu.VMEM((stripe, width), jnp.float32),
          pltpu.VMEM((stripe, width), jnp.float32),
          pltpu.VMEM((stripe, width), jnp.float32),
          pltpu.VMEM((stripe, width), jnp.float32),
          pltpu.VMEM((1, width), jnp.float32),
          pltpu.VMEM_SHARED((n_pad, width), jnp.float32),
          pltpu.VMEM_SHARED((n_pad, width), jnp.float32),
          pltpu.SemaphoreType.DMA,
          pltpu.SemaphoreType.DMA,
      ],
      compiler_params=_sc_params,
  )
  def agg2_kernel(p_hbm, hs_hbm, dis_hbm, b1_hbm, ridx_hbm, cidx_hbm,
                  zeros_hbm, out_hbm, h1s_hbm,
                  ridx_v, cidx_v, buf0, buf1, p0v, p1v, hsv, disv, ov, b1v,
                  tbl, acc, sem0, sem1):
    cid = lax.axis_index("c")
    sid = lax.axis_index("s")
    wid = sid * NC + cid
    base = sid * stripe
    pltpu.sync_copy(ridx_hbm.at[wid], ridx_v)
    pltpu.sync_copy(cidx_hbm.at[wid], cidx_v)
    pltpu.sync_copy(p_hbm.at[0, pl.ds(base, stripe)], p0v)
    pltpu.sync_copy(p_hbm.at[1, pl.ds(base, stripe)], p1v)
    pltpu.sync_copy(hs_hbm.at[pl.ds(base, stripe)], hsv)
    pltpu.sync_copy(dis_hbm.at[pl.ds(base, stripe)], disv)
    pltpu.sync_copy(b1_hbm, b1v)
    b1r = b1v[0, :]

    def erow(r, carry):
      dr = disv[r, :]
      v = (p0v[r, :] + p1v[r, :] + hsv[r, :]) * dr + b1r
      ov[r, :] = jnp.maximum(v, 0.0) * dr
      return carry

    lax.fori_loop(0, stripe, erow, 0)
    pltpu.sync_copy(ov, tbl.at[pl.ds(base, stripe)])

    @pl.when(cid == 0)
    def _():
      pltpu.sync_copy(ov, h1s_hbm.at[pl.ds(base, stripe)])

    pltpu.sync_copy(zeros_hbm, ov)
    pltpu.sync_copy(ov, acc.at[pl.ds(base, stripe)])
    plsc.subcore_barrier()
    _agg_loop(tbl, ridx_v, cidx_v, acc, buf0, buf1, sem0, sem1, nchunk)
    plsc.subcore_barrier()
    pltpu.sync_copy(acc.at[pl.ds(base, stripe)],
                    out_hbm.at[cid, pl.ds(base, stripe)])

  return agg2_kernel


def _tc_a_body(deg_ref, x_ref, w1_ref, hs_ref, dis_ref):
  deg = deg_ref[0, :, :1] + deg_ref[1, :, :1] + 1.0
  dis = lax.rsqrt(deg)
  h = jnp.dot(x_ref[...], w1_ref[...], preferred_element_type=jnp.float32)
  hs_ref[...] = h * dis
  dis_ref[...] = jnp.broadcast_to(dis, dis_ref.shape)


def _tc_c_body(p_ref, hs1_ref, dis_ref, w2_ref, b2_ref, out_ref):
  agg = (p_ref[0] + p_ref[1] + hs1_ref[...]) * dis_ref[...]
  o = jnp.dot(agg, w2_ref[...], preferred_element_type=jnp.float32)
  o = o + b2_ref[...]
  m = jnp.max(o, axis=1, keepdims=True)
  lse = jnp.log(jnp.sum(jnp.exp(o - m), axis=1, keepdims=True)) + m
  out_ref[...] = o - lse


def kernel(x, edge_index, W1, b1, W2, b2):
  n, d = x.shape
  hid = W1.shape[1]
  c = W2.shape[1]
  e = edge_index.shape[1]

  blk = 640
  n_pad = pl.cdiv(n + 1, blk) * blk
  stripe = n_pad // NS
  ept = pl.cdiv(e, NT)
  nchunk = pl.cdiv(ept, L)
  nchunk += nchunk % 2
  e_pad = NT * nchunk * L
  nblk = n_pad // blk

  row = jnp.pad(edge_index[0], (0, e_pad - e), constant_values=n)
  col = jnp.pad(edge_index[1], (0, e_pad - e), constant_values=n)
  ridx = row.reshape(NT, nchunk, L)
  cidx = col.reshape(NT, nchunk, L)
  x_p = jnp.pad(x, ((0, n_pad - n), (0, 0)))
  ones_l = jnp.ones((L, 8), jnp.float32)
  zeros_s8 = jnp.zeros((stripe, 8), jnp.float32)
  zeros_sh = jnp.zeros((stripe, hid), jnp.float32)

  deg_p = _make_deg_kernel(n_pad, nchunk, stripe, 8)(cidx, ones_l, zeros_s8)

  hs, dis = pl.pallas_call(
      _tc_a_body,
      grid=(nblk,),
      in_specs=[
          pl.BlockSpec((NC, blk, 8), lambda i: (0, i, 0)),
          pl.BlockSpec((blk, d), lambda i: (i, 0)),
          pl.BlockSpec((d, hid), lambda i: (0, 0)),
      ],
      out_specs=[
          pl.BlockSpec((blk, hid), lambda i: (i, 0)),
          pl.BlockSpec((blk, hid), lambda i: (i, 0)),
      ],
      out_shape=[
          jax.ShapeDtypeStruct((n_pad, hid), jnp.float32),
          jax.ShapeDtypeStruct((n_pad, hid), jnp.float32),
      ],
  )(deg_p, x_p, W1)

  p1 = _make_agg1_kernel(n_pad, nchunk, stripe, hid)(hs, ridx, cidx, zeros_sh)

  p2, hs1 = _make_agg2_kernel(n_pad, nchunk, stripe, hid)(
      p1, hs, dis, b1.reshape(1, hid), ridx, cidx, zeros_sh)

  out = pl.pallas_call(
      _tc_c_body,
      grid=(nblk,),
      in_specs=[
          pl.BlockSpec((NC, blk, hid), lambda i: (0, i, 0)),
          pl.BlockSpec((blk, hid), lambda i: (i, 0)),
          pl.BlockSpec((blk, hid), lambda i: (i, 0)),
          pl.BlockSpec((hid, c), lambda i: (0, 0)),
          pl.BlockSpec((1, c), lambda i: (0, 0)),
      ],
      out_specs=pl.BlockSpec((blk, c), lambda i: (i, 0)),
      out_shape=jax.ShapeDtypeStruct((n_pad, c), jnp.float32),
  )(p2, hs1, dis, W2, b2.reshape(1, c))

  return out[:n]


# speedup vs baseline: 57.2742x; 1.0884x over previous
import functools

import jax
import jax.numpy as jnp
from jax import lax
from jax.experimental import pallas as pl
from jax.experimental.pallas import tpu as pltpu
from jax.experimental.pallas import tpu_sc as plsc

NC = 2
NS = 16
L = 512
NT = NC * NS

_mesh = functools.partial(
    plsc.VectorSubcoreMesh,
    core_axis_name="c",
    subcore_axis_name="s",
    num_cores=NC,
    num_subcores=NS,
)
_sc_params = pltpu.CompilerParams(use_tc_tiling_on_sc=False)


def _load_slab(edge_hbm, which, idx_v, wid, ept, eptp, n):
  pltpu.sync_copy(edge_hbm.at[which, pl.ds(wid * ept, ept)],
                  idx_v.at[pl.ds(0, ept)])
  pad = jnp.full((16,), n, jnp.int32)
  for k in range((eptp - ept) // 16):
    idx_v[pl.ds(ept + k * 16, 16)] = pad


def _agg_loop(tbl, ridx_v, cidx_v, acc, buf0, buf1, sem0, sem1, nchunk):
  pltpu.async_copy(tbl.at[ridx_v.at[pl.ds(0, L)]], buf0, sem0)

  def body(k, carry):
    j0 = 2 * k
    j1 = j0 + 1
    pltpu.make_async_copy(tbl.at[pl.ds(0, L)], buf0, sem0).wait()
    pltpu.async_copy(tbl.at[ridx_v.at[pl.ds(j1 * L, L)]], buf1, sem1)
    pltpu.sync_copy(buf0, acc.at[cidx_v.at[pl.ds(j0 * L, L)]], add=True)
    jn = jnp.minimum(j0 + 2, nchunk - 2) * L
    pltpu.async_copy(tbl.at[ridx_v.at[pl.ds(jn, L)]], buf0, sem0)
    pltpu.make_async_copy(tbl.at[pl.ds(0, L)], buf1, sem1).wait()
    pltpu.sync_copy(buf1, acc.at[cidx_v.at[pl.ds(j1 * L, L)]], add=True)
    return carry

  lax.fori_loop(0, nchunk // 2, body, 0)
  pltpu.make_async_copy(tbl.at[pl.ds(0, L)], buf0, sem0).wait()


def _make_deg_kernel(n_pad, ept, eptp, stripe, width, n):
  @functools.partial(
      pl.kernel,
      mesh=_mesh(),
      out_type=jax.ShapeDtypeStruct((NC, n_pad, width), jnp.float32),
      scratch_types=[
          pltpu.VMEM((eptp,), jnp.int32),
          pltpu.VMEM((L, width), jnp.float32),
          pltpu.VMEM((stripe, width), jnp.float32),
          pltpu.VMEM_SHARED((n_pad, width), jnp.float32),
          pltpu.SemaphoreType.DMA,
      ],
      compiler_params=_sc_params,
  )
  def deg_kernel(edge_hbm, ones_hbm, zeros_hbm, out_hbm,
                 cidx_v, ones_v, zer_v, acc, sem):
    cid = lax.axis_index("c")
    sid = lax.axis_index("s")
    wid = sid * NC + cid
    _load_slab(edge_hbm, 1, cidx_v, wid, ept, eptp, n)
    pltpu.sync_copy(ones_hbm, ones_v)
    pltpu.sync_copy(zeros_hbm, zer_v)
    pltpu.sync_copy(zer_v, acc.at[pl.ds(sid * stripe, stripe)])
    plsc.subcore_barrier()

    def body(j, carry):
      pltpu.sync_copy(ones_v, acc.at[cidx_v.at[pl.ds(j * L, L)]], add=True)
      return carry

    lax.fori_loop(0, eptp // L, body, 0)
    plsc.subcore_barrier()
    pltpu.sync_copy(acc.at[pl.ds(sid * stripe, stripe)],
                    out_hbm.at[cid, pl.ds(sid * stripe, stripe)])

  return deg_kernel


def _make_agg1_kernel(n_pad, ept, eptp, stripe, width, n):
  @functools.partial(
      pl.kernel,
      mesh=_mesh(),
      out_type=jax.ShapeDtypeStruct((NC, n_pad, width), jnp.float32),
      scratch_types=[
          pltpu.VMEM((eptp,), jnp.int32),
          pltpu.VMEM((eptp,), jnp.int32),
          pltpu.VMEM((L, width), jnp.float32),
          pltpu.VMEM((L, width), jnp.float32),
          pltpu.VMEM((stripe, width), jnp.float32),
          pltpu.VMEM_SHARED((n_pad, width), jnp.float32),
          pltpu.VMEM_SHARED((n_pad, width), jnp.float32),
          pltpu.SemaphoreType.DMA,
          pltpu.SemaphoreType.DMA,
      ],
      compiler_params=_sc_params,
  )
  def agg1_kernel(table_hbm, edge_hbm, zeros_hbm, out_hbm,
                  ridx_v, cidx_v, buf0, buf1, zer_v, tbl, acc, sem0, sem1):
    cid = lax.axis_index("c")
    sid = lax.axis_index("s")
    wid = sid * NC + cid
    base = sid * stripe
    _load_slab(edge_hbm, 0, ridx_v, wid, ept, eptp, n)
    _load_slab(edge_hbm, 1, cidx_v, wid, ept, eptp, n)
    pltpu.sync_copy(zeros_hbm, zer_v)
    pltpu.sync_copy(zer_v, acc.at[pl.ds(base, stripe)])
    pltpu.sync_copy(table_hbm.at[pl.ds(base, stripe)],
                    tbl.at[pl.ds(base, stripe)])
    plsc.subcore_barrier()
    _agg_loop(tbl, ridx_v, cidx_v, acc, buf0, buf1, sem0, sem1, eptp // L)
    plsc.subcore_barrier()
    pltpu.sync_copy(acc.at[pl.ds(base, stripe)],
                    out_hbm.at[cid, pl.ds(base, stripe)])

  return agg1_kernel


def _make_agg2_kernel(n_pad, ept, eptp, stripe, width, n):
  @functools.partial(
      pl.kernel,
      mesh=_mesh(),
      out_type=(
          jax.ShapeDtypeStruct((NC, n_pad, width), jnp.float32),
          jax.ShapeDtypeStruct((n_pad, width), jnp.float32),
      ),
      scratch_types=[
          pltpu.VMEM((eptp,), jnp.int32),
          pltpu.VMEM((eptp,), jnp.int32),
          pltpu.VMEM((L, width), jnp.float32),
          pltpu.VMEM((L, width), jnp.float32),
          pltpu.VMEM((stripe, width), jnp.float32),
          pltpu.VMEM((stripe, width), jnp.float32),
          pltpu.VMEM((stripe, width), jnp.float32),
          pltpu.VMEM((stripe, width), jnp.float32),
          pltpu.VMEM((stripe, width), jnp.float32),
          pltpu.VMEM((1, width), jnp.float32),
          pltpu.VMEM_SHARED((n_pad, width), jnp.float32),
          pltpu.VMEM_SHARED((n_pad, width), jnp.float32),
          pltpu.SemaphoreType.DMA,
          pltpu.SemaphoreType.DMA,
      ],
      compiler_params=_sc_params,
  )
  def agg2_kernel(p_hbm, hs_hbm, dis_hbm, b1_hbm, edge_hbm,
                  zeros_hbm, out_hbm, h1s_hbm,
                  ridx_v, cidx_v, buf0, buf1, p0v, p1v, hsv, disv, ov, b1v,
                  tbl, acc, sem0, sem1):
    cid = lax.axis_index("c")
    sid = lax.axis_index("s")
    wid = sid * NC + cid
    base = sid * stripe
    _load_slab(edge_hbm, 0, ridx_v, wid, ept, eptp, n)
    _load_slab(edge_hbm, 1, cidx_v, wid, ept, eptp, n)
    pltpu.sync_copy(p_hbm.at[0, pl.ds(base, stripe)], p0v)
    pltpu.sync_copy(p_hbm.at[1, pl.ds(base, stripe)], p1v)
    pltpu.sync_copy(hs_hbm.at[pl.ds(base, stripe)], hsv)
    pltpu.sync_copy(dis_hbm.at[pl.ds(base, stripe)], disv)
    pltpu.sync_copy(b1_hbm, b1v)
    b1r = b1v[0, :]

    def erow(r, carry):
      dr = disv[r, :]
      v = (p0v[r, :] + p1v[r, :] + hsv[r, :]) * dr + b1r
      ov[r, :] = jnp.maximum(v, 0.0) * dr
      return carry

    lax.fori_loop(0, stripe, erow, 0)
    pltpu.sync_copy(ov, tbl.at[pl.ds(base, stripe)])

    @pl.when(cid == 0)
    def _():
      pltpu.sync_copy(ov, h1s_hbm.at[pl.ds(base, stripe)])

    pltpu.sync_copy(zeros_hbm, ov)
    pltpu.sync_copy(ov, acc.at[pl.ds(base, stripe)])
    plsc.subcore_barrier()
    _agg_loop(tbl, ridx_v, cidx_v, acc, buf0, buf1, sem0, sem1, eptp // L)
    plsc.subcore_barrier()
    pltpu.sync_copy(acc.at[pl.ds(base, stripe)],
                    out_hbm.at[cid, pl.ds(base, stripe)])

  return agg2_kernel


def _tc_a_body(deg_ref, x_ref, w1_ref, hs_ref, dis_ref):
  dis = lax.rsqrt(deg_ref[0] + deg_ref[1] + 1.0)
  h = jnp.dot(x_ref[...], w1_ref[...], preferred_element_type=jnp.float32)
  hs_ref[...] = h * dis
  dis_ref[...] = dis


def _tc_c_body(p_ref, hs1_ref, dis_ref, w2_ref, b2_ref, out_ref):
  agg = (p_ref[0] + p_ref[1] + hs1_ref[...]) * dis_ref[...]
  o = jnp.dot(agg, w2_ref[...], preferred_element_type=jnp.float32)
  o = o + b2_ref[...]
  m = jnp.max(o, axis=1, keepdims=True)
  lse = jnp.log(jnp.sum(jnp.exp(o - m), axis=1, keepdims=True)) + m
  out_ref[...] = o - lse


def kernel(x, edge_index, W1, b1, W2, b2):
  n, d = x.shape
  hid = W1.shape[1]
  c = W2.shape[1]
  e = edge_index.shape[1]

  blk = 640
  n_pad = pl.cdiv(n + 1, blk) * blk
  stripe = n_pad // NS
  ept = e // NT
  assert ept * NT == e and ept % 8 == 0
  nchunk = pl.cdiv(ept, L)
  nchunk += nchunk % 2
  eptp = nchunk * L
  nblk = n_pad // blk

  ones_l = jnp.ones((L, hid), jnp.float32)
  zeros_sh = jnp.zeros((stripe, hid), jnp.float32)

  deg_p = _make_deg_kernel(n_pad, ept, eptp, stripe, hid, n)(
      edge_index, ones_l, zeros_sh)

  hs, dis = pl.pallas_call(
      _tc_a_body,
      grid=(nblk,),
      in_specs=[
          pl.BlockSpec((NC, blk, hid), lambda i: (0, i, 0)),
          pl.BlockSpec((blk, d), lambda i: (i, 0)),
          pl.BlockSpec((d, hid), lambda i: (0, 0)),
      ],
      out_specs=[
          pl.BlockSpec((blk, hid), lambda i: (i, 0)),
          pl.BlockSpec((blk, hid), lambda i: (i, 0)),
      ],
      out_shape=[
          jax.ShapeDtypeStruct((n_pad, hid), jnp.float32),
          jax.ShapeDtypeStruct((n_pad, hid), jnp.float32),
      ],
  )(deg_p, x, W1)

  p1 = _make_agg1_kernel(n_pad, ept, eptp, stripe, hid, n)(
      hs, edge_index, zeros_sh)

  p2, hs1 = _make_agg2_kernel(n_pad, ept, eptp, stripe, hid, n)(
      p1, hs, dis, b1.reshape(1, hid), edge_index, zeros_sh)

  out = pl.pallas_call(
      _tc_c_body,
      grid=(nblk,),
      in_specs=[
          pl.BlockSpec((NC, blk, hid), lambda i: (0, i, 0)),
          pl.BlockSpec((blk, hid), lambda i: (i, 0)),
          pl.BlockSpec((blk, hid), lambda i: (i, 0)),
          pl.BlockSpec((hid, c), lambda i: (0, 0)),
          pl.BlockSpec((1, c), lambda i: (0, 0)),
      ],
      out_specs=pl.BlockSpec((blk, c), lambda i: (i, 0)),
      out_shape=jax.ShapeDtypeStruct((n_pad, c), jnp.float32),
  )(p2, hs1, dis, W2, b2.reshape(1, c))

  return out[:n]


# speedup vs baseline: 61.2251x; 1.0690x over previous
import functools

import jax
import jax.numpy as jnp
from jax import lax
from jax.experimental import pallas as pl
from jax.experimental.pallas import tpu as pltpu
from jax.experimental.pallas import tpu_sc as plsc

NC = 2
NS = 16
L = 512
NT = NC * NS

_mesh = functools.partial(
    plsc.VectorSubcoreMesh,
    core_axis_name="c",
    subcore_axis_name="s",
    num_cores=NC,
    num_subcores=NS,
)
_sc_params = pltpu.CompilerParams(use_tc_tiling_on_sc=False)


def _load_slab(edge_hbm, which, idx_v, wid, ept, eptp, n):
  pltpu.sync_copy(edge_hbm.at[which, pl.ds(wid * ept, ept)],
                  idx_v.at[pl.ds(0, ept)])
  pad = jnp.full((16,), n, jnp.int32)
  for k in range((eptp - ept) // 16):
    idx_v[pl.ds(ept + k * 16, 16)] = pad


def _agg_loop(tbl, ridx_v, cidx_v, acc, buf0, buf1, sem0, sem1, nchunk):
  pltpu.async_copy(tbl.at[ridx_v.at[pl.ds(0, L)]], buf0, sem0)

  def body(k, carry):
    j0 = 2 * k
    j1 = j0 + 1
    pltpu.make_async_copy(tbl.at[pl.ds(0, L)], buf0, sem0).wait()
    pltpu.async_copy(tbl.at[ridx_v.at[pl.ds(j1 * L, L)]], buf1, sem1)
    pltpu.sync_copy(buf0, acc.at[cidx_v.at[pl.ds(j0 * L, L)]], add=True)
    jn = jnp.minimum(j0 + 2, nchunk - 2) * L
    pltpu.async_copy(tbl.at[ridx_v.at[pl.ds(jn, L)]], buf0, sem0)
    pltpu.make_async_copy(tbl.at[pl.ds(0, L)], buf1, sem1).wait()
    pltpu.sync_copy(buf1, acc.at[cidx_v.at[pl.ds(j1 * L, L)]], add=True)
    return carry

  lax.fori_loop(0, nchunk // 2, body, 0)
  pltpu.make_async_copy(tbl.at[pl.ds(0, L)], buf0, sem0).wait()


def _make_deg_kernel(n_pad, ept, eptp, stripe, width, n):
  @functools.partial(
      pl.kernel,
      mesh=_mesh(),
      out_type=jax.ShapeDtypeStruct((NC, n_pad, width), jnp.float32),
      scratch_types=[
          pltpu.VMEM((eptp,), jnp.int32),
          pltpu.VMEM((L, width), jnp.float32),
          pltpu.VMEM((stripe, width), jnp.float32),
          pltpu.VMEM_SHARED((n_pad, width), jnp.float32),
          pltpu.SemaphoreType.DMA,
      ],
      compiler_params=_sc_params,
  )
  def deg_kernel(edge_hbm, ones_hbm, zeros_hbm, out_hbm,
                 cidx_v, ones_v, zer_v, acc, sem):
    cid = lax.axis_index("c")
    sid = lax.axis_index("s")
    wid = sid * NC + cid
    _load_slab(edge_hbm, 1, cidx_v, wid, ept, eptp, n)
    pltpu.sync_copy(ones_hbm, ones_v)
    pltpu.sync_copy(zeros_hbm, zer_v)
    pltpu.sync_copy(zer_v, acc.at[pl.ds(sid * stripe, stripe)])
    plsc.subcore_barrier()

    def body(j, carry):
      pltpu.sync_copy(ones_v, acc.at[cidx_v.at[pl.ds(j * L, L)]], add=True)
      return carry

    lax.fori_loop(0, eptp // L, body, 0)
    plsc.subcore_barrier()
    pltpu.sync_copy(acc.at[pl.ds(sid * stripe, stripe)],
                    out_hbm.at[cid, pl.ds(sid * stripe, stripe)])

  return deg_kernel


def _make_agg1_kernel(n_pad, ept, eptp, stripe, width, n):
  @functools.partial(
      pl.kernel,
      mesh=_mesh(),
      out_type=jax.ShapeDtypeStruct((NC, n_pad, width), jnp.float32),
      scratch_types=[
          pltpu.VMEM((eptp,), jnp.int32),
          pltpu.VMEM((eptp,), jnp.int32),
          pltpu.VMEM((L, width), jnp.float32),
          pltpu.VMEM((L, width), jnp.float32),
          pltpu.VMEM((stripe, width), jnp.float32),
          pltpu.VMEM_SHARED((n_pad, width), jnp.float32),
          pltpu.VMEM_SHARED((n_pad, width), jnp.float32),
          pltpu.SemaphoreType.DMA,
          pltpu.SemaphoreType.DMA,
      ],
      compiler_params=_sc_params,
  )
  def agg1_kernel(table_hbm, edge_hbm, zeros_hbm, out_hbm,
                  ridx_v, cidx_v, buf0, buf1, zer_v, tbl, acc, sem0, sem1):
    cid = lax.axis_index("c")
    sid = lax.axis_index("s")
    wid = sid * NC + cid
    base = sid * stripe
    _load_slab(edge_hbm, 0, ridx_v, wid, ept, eptp, n)
    _load_slab(edge_hbm, 1, cidx_v, wid, ept, eptp, n)
    pltpu.sync_copy(zeros_hbm, zer_v)
    pltpu.sync_copy(zer_v, acc.at[pl.ds(base, stripe)])
    pltpu.sync_copy(table_hbm.at[pl.ds(base, stripe)],
                    tbl.at[pl.ds(base, stripe)])
    plsc.subcore_barrier()
    _agg_loop(tbl, ridx_v, cidx_v, acc, buf0, buf1, sem0, sem1, eptp // L)
    plsc.subcore_barrier()
    pltpu.sync_copy(acc.at[pl.ds(base, stripe)],
                    out_hbm.at[cid, pl.ds(base, stripe)])

  return agg1_kernel


def _make_agg2_kernel(n_pad, ept, eptp, stripe, width, n):
  @functools.partial(
      pl.kernel,
      mesh=_mesh(),
      out_type=(
          jax.ShapeDtypeStruct((NC, n_pad, width), jnp.float32),
          jax.ShapeDtypeStruct((n_pad, width), jnp.float32),
      ),
      scratch_types=[
          pltpu.VMEM((eptp,), jnp.int32),
          pltpu.VMEM((eptp,), jnp.int32),
          pltpu.VMEM((L, width), jnp.float32),
          pltpu.VMEM((L, width), jnp.float32),
          pltpu.VMEM((stripe, width), jnp.float32),
          pltpu.VMEM((stripe, width), jnp.float32),
          pltpu.VMEM((stripe, width), jnp.float32),
          pltpu.VMEM((stripe, width), jnp.float32),
          pltpu.VMEM((stripe, width), jnp.float32),
          pltpu.VMEM((1, width), jnp.float32),
          pltpu.VMEM_SHARED((n_pad, width), jnp.float32),
          pltpu.VMEM_SHARED((n_pad, width), jnp.float32),
          pltpu.SemaphoreType.DMA,
          pltpu.SemaphoreType.DMA,
      ],
      compiler_params=_sc_params,
  )
  def agg2_kernel(p_hbm, hs_hbm, dis_hbm, b1_hbm, edge_hbm,
                  zeros_hbm, out_hbm, h1s_hbm,
                  ridx_v, cidx_v, buf0, buf1, p0v, p1v, hsv, disv, ov, b1v,
                  tbl, acc, sem0, sem1):
    cid = lax.axis_index("c")
    sid = lax.axis_index("s")
    wid = sid * NC + cid
    base = sid * stripe
    _load_slab(edge_hbm, 0, ridx_v, wid, ept, eptp, n)
    _load_slab(edge_hbm, 1, cidx_v, wid, ept, eptp, n)
    pltpu.sync_copy(p_hbm.at[0, pl.ds(base, stripe)], p0v)
    pltpu.sync_copy(p_hbm.at[1, pl.ds(base, stripe)], p1v)
    pltpu.sync_copy(hs_hbm.at[pl.ds(base, stripe)], hsv)
    pltpu.sync_copy(dis_hbm.at[pl.ds(base, stripe)], disv)
    pltpu.sync_copy(b1_hbm, b1v)
    b1r = b1v[0, :]

    def erow(r, carry):
      dr = disv[r, :]
      v = (p0v[r, :] + p1v[r, :] + hsv[r, :]) * dr + b1r
      ov[r, :] = jnp.maximum(v, 0.0) * dr
      return carry

    lax.fori_loop(0, stripe, erow, 0)
    pltpu.sync_copy(ov, tbl.at[pl.ds(base, stripe)])

    @pl.when(cid == 0)
    def _():
      pltpu.sync_copy(ov, h1s_hbm.at[pl.ds(base, stripe)])

    pltpu.sync_copy(zeros_hbm, ov)
    pltpu.sync_copy(ov, acc.at[pl.ds(base, stripe)])
    plsc.subcore_barrier()
    _agg_loop(tbl, ridx_v, cidx_v, acc, buf0, buf1, sem0, sem1, eptp // L)
    plsc.subcore_barrier()
    pltpu.sync_copy(acc.at[pl.ds(base, stripe)],
                    out_hbm.at[cid, pl.ds(base, stripe)])

  return agg2_kernel


def _tc_a_body(deg_ref, x_ref, w1_ref, hs_ref, dis_ref):
  dis = lax.rsqrt(deg_ref[0] + deg_ref[1] + 1.0)
  h = jnp.dot(x_ref[...], w1_ref[...], preferred_element_type=jnp.float32)
  hs_ref[...] = h * dis
  dis_ref[...] = dis


def _tc_c_body(p_ref, hs1_ref, dis_ref, w2_ref, b2_ref, out_ref):
  agg = (p_ref[0] + p_ref[1] + hs1_ref[...]) * dis_ref[...]
  o = jnp.dot(agg, w2_ref[...], preferred_element_type=jnp.float32)
  o = o + b2_ref[...]
  m = jnp.max(o, axis=1, keepdims=True)
  lse = jnp.log(jnp.sum(jnp.exp(o - m), axis=1, keepdims=True)) + m
  out_ref[...] = o - lse


def kernel(x, edge_index, W1, b1, W2, b2):
  n, d = x.shape
  hid = W1.shape[1]
  c = W2.shape[1]
  e = edge_index.shape[1]

  blk = 640
  n_pad = pl.cdiv(n + 1, blk) * blk
  stripe = n_pad // NS
  ept = e // NT
  assert ept * NT == e and ept % 8 == 0
  nchunk = pl.cdiv(ept, L)
  nchunk += nchunk % 2
  eptp = nchunk * L
  nblk = n_pad // blk

  ones_l = jnp.ones((L, hid), jnp.float32)
  zeros_sh = jnp.zeros((stripe, hid), jnp.float32)

  deg_p = _make_deg_kernel(n_pad, ept, eptp, stripe, hid, n)(
      edge_index, ones_l, zeros_sh)

  hs, dis = pl.pallas_call(
      _tc_a_body,
      grid=(1,),
      in_specs=[
          pl.BlockSpec((NC, n_pad, hid), lambda i: (0, 0, 0)),
          pl.BlockSpec((n_pad, d), lambda i: (0, 0)),
          pl.BlockSpec((d, hid), lambda i: (0, 0)),
      ],
      out_specs=[
          pl.BlockSpec((n_pad, hid), lambda i: (0, 0)),
          pl.BlockSpec((n_pad, hid), lambda i: (0, 0)),
      ],
      out_shape=[
          jax.ShapeDtypeStruct((n_pad, hid), jnp.float32),
          jax.ShapeDtypeStruct((n_pad, hid), jnp.float32),
      ],
  )(deg_p, x, W1)

  p1 = _make_agg1_kernel(n_pad, ept, eptp, stripe, hid, n)(
      hs, edge_index, zeros_sh)

  p2, hs1 = _make_agg2_kernel(n_pad, ept, eptp, stripe, hid, n)(
      p1, hs, dis, b1.reshape(1, hid), edge_index, zeros_sh)

  out = pl.pallas_call(
      _tc_c_body,
      grid=(1,),
      in_specs=[
          pl.BlockSpec((NC, n, hid), lambda i: (0, 0, 0)),
          pl.BlockSpec((n, hid), lambda i: (0, 0)),
          pl.BlockSpec((n, hid), lambda i: (0, 0)),
          pl.BlockSpec((hid, c), lambda i: (0, 0)),
          pl.BlockSpec((1, c), lambda i: (0, 0)),
      ],
      out_specs=pl.BlockSpec((n, c), lambda i: (0, 0)),
      out_shape=jax.ShapeDtypeStruct((n, c), jnp.float32),
  )(p2, hs1, dis, W2, b2.reshape(1, c))

  return out
